# Initial kernel scaffold; baseline (speedup 1.0000x reference)
import jax
import jax.numpy as jnp
from jax.experimental import pallas as pl


def kernel(x, edge_index, edge_attr, ptr, W1, b1, W2, b2, W3, b3, Wl, bl):
    raise NotImplementedError("write your pallas kernel here")


# speedup vs baseline: 9.7697x; 9.7697x over previous
import functools

import jax
import jax.numpy as jnp
from jax import lax
from jax.experimental import pallas as pl
from jax.experimental.pallas import tpu as pltpu
from jax.experimental.pallas import tpu_sc as plsc

N = 10000
E = 320000
D = 128
OUTD = 64
NG = 16

NC, NS, L = 2, 16, 16
NW = NC * NS
NPAD = 10240
NROW = 80
EPW = E // NW
EPT = E // NS
CH = 80
NCH = EPW // CH

_MESH = plsc.VectorSubcoreMesh(
    core_axis_name="c", subcore_axis_name="s", num_cores=NC, num_subcores=NS)


def _rsqrt16(d):
    i = plsc.bitcast(d, jnp.int32)
    y = plsc.bitcast(jnp.int32(0x5F3759DF) - (i >> 1), jnp.float32)
    for _ in range(3):
        y = y * (1.5 - 0.5 * d * y * y)
    return y


def _prep_body(row_hbm, col_hbm, ew_hbm,
               norm_hbm, dinv2_hbm,
               colb, ewb, rowb, normb, degacc, dsum, tmp, yloc, y2loc,
               sdeg_all, sdinv):
    c = lax.axis_index("c")
    s = lax.axis_index("s")
    wid = c * NS + s
    zeros16 = jnp.zeros((16,), jnp.float32)
    npt = NPAD // NS

    @pl.loop(0, NPAD // 16)
    def _(r):
        degacc[pl.ds(r * 16, 16)] = zeros16

    for half in range(2):
        off = s * EPT + half * EPW
        pltpu.sync_copy(col_hbm.at[pl.ds(off, EPW)], colb)
        pltpu.sync_copy(ew_hbm.at[pl.ds(off, EPW)], ewb)

        @pl.loop(0, EPW // 16)
        def _(i):
            c16 = colb[pl.ds(i * 16, 16)]
            w16 = ewb[pl.ds(i * 16, 16)]
            plsc.addupdate_scatter(degacc, [c16], w16)

    pltpu.sync_copy(degacc, sdeg_all.at[s])
    plsc.subcore_barrier()

    @pl.loop(0, npt // 16)
    def _(j):
        dsum[pl.ds(j * 16, 16)] = zeros16
    for p in range(NS):
        pltpu.sync_copy(sdeg_all.at[p, pl.ds(s * npt, npt)], tmp)

        @pl.loop(0, npt // 16)
        def _(j):
            sl = pl.ds(j * 16, 16)
            dsum[sl] = dsum[sl] + tmp[sl]

    @pl.loop(0, npt // 16)
    def _(j):
        sl = pl.ds(j * 16, 16)
        d = dsum[sl] + 1.0
        y = _rsqrt16(d)
        yloc[sl] = y
        y2loc[sl] = y * y

    pltpu.sync_copy(yloc, sdinv.at[pl.ds(s * npt, npt)])

    @pl.when(c == 0)
    def _():
        pltpu.sync_copy(y2loc, dinv2_hbm.at[pl.ds(s * npt, npt)])

    plsc.subcore_barrier()

    pltpu.sync_copy(sdinv, degacc)
    off = wid * EPW
    pltpu.sync_copy(row_hbm.at[pl.ds(off, EPW)], rowb)
    pltpu.sync_copy(col_hbm.at[pl.ds(off, EPW)], colb)
    pltpu.sync_copy(ew_hbm.at[pl.ds(off, EPW)], ewb)

    @pl.loop(0, EPW // 16)
    def _(i):
        sl = pl.ds(i * 16, 16)
        dr = plsc.load_gather(degacc, [rowb[sl]])
        dc = plsc.load_gather(degacc, [colb[sl]])
        normb[sl] = dr * dc * ewb[sl]

    pltpu.sync_copy(normb, norm_hbm.at[pl.ds(off, EPW)])


_prep = pl.kernel(
    _prep_body,
    out_type=(
        jax.ShapeDtypeStruct((E,), jnp.float32),
        jax.ShapeDtypeStruct((NPAD,), jnp.float32),
    ),
    mesh=_MESH,
    scratch_types=[
        pltpu.VMEM((EPW,), jnp.int32),
        pltpu.VMEM((EPW,), jnp.float32),
        pltpu.VMEM((EPW,), jnp.int32),
        pltpu.VMEM((EPW,), jnp.float32),
        pltpu.VMEM((NPAD,), jnp.float32),
        pltpu.VMEM((NPAD // NS,), jnp.float32),
        pltpu.VMEM((NPAD // NS,), jnp.float32),
        pltpu.VMEM((NPAD // NS,), jnp.float32),
        pltpu.VMEM((NPAD // NS,), jnp.float32),
        pltpu.VMEM_SHARED((NS, NPAD), jnp.float32),
        pltpu.VMEM_SHARED((NPAD,), jnp.float32),
    ],
    compiler_params=pltpu.CompilerParams(needs_layout_passes=False),
)


def _mp_body(xw_hbm, row_hbm, col_hbm, norm_hbm,
             parts_hbm,
             rowb, normb, rows, cidx, zbuf,
             acc):
    c = lax.axis_index("c")
    s = lax.axis_index("s")
    wid = c * NS + s
    zeros16 = jnp.zeros((16,), jnp.float32)

    @pl.loop(0, NROW)
    def _(r):
        for j in range(8):
            zbuf[r, pl.ds(j * 16, 16)] = zeros16
    for k in range(8):
        pltpu.sync_copy(zbuf, acc.at[pl.ds(s * 640 + k * 80, 80)])
    plsc.subcore_barrier()

    base = wid * EPW
    pltpu.sync_copy(row_hbm.at[pl.ds(base, EPW)], rowb)
    pltpu.sync_copy(norm_hbm.at[pl.ds(base, EPW)], normb)

    @pl.loop(0, NCH)
    def _(g):
        off = g * CH
        pltpu.sync_copy(xw_hbm.at[rowb.at[pl.ds(off, CH)]], rows)
        for grp in range(5):
            for e in range(16):
                ei = grp * 16 + e
                ne = plsc.load_gather(
                    normb, [jnp.full((16,), ei, jnp.int32) + off])
                for j in range(8):
                    sl = pl.ds(j * 16, 16)
                    rows[ei, sl] = rows[ei, sl] * ne
        pltpu.sync_copy(col_hbm.at[pl.ds(base + off, CH)], cidx)
        pltpu.sync_copy(rows, acc.at[cidx], add=True)

    plsc.subcore_barrier()
    pltpu.sync_copy(acc.at[pl.ds(s * 640, 640)],
                    parts_hbm.at[c, pl.ds(s * 640, 640)])


_mp = pl.kernel(
    _mp_body,
    out_type=jax.ShapeDtypeStruct((NC, NPAD, 128), jnp.float32),
    mesh=_MESH,
    scratch_types=[
        pltpu.VMEM((EPW,), jnp.int32),
        pltpu.VMEM((EPW,), jnp.float32),
        pltpu.VMEM((CH, 128), jnp.float32),
        pltpu.VMEM((CH,), jnp.int32),
        pltpu.VMEM((NROW, 128), jnp.float32),
        pltpu.VMEM_SHARED((NPAD, 128), jnp.float32),
    ],
    compiler_params=pltpu.CompilerParams(needs_layout_passes=False),
)


def _mm_body(x_ref, w_ref, o_ref):
    o_ref[...] = jnp.dot(x_ref[...], w_ref[...],
                         preferred_element_type=jnp.float32)


def _mm(xpad, W):
    return pl.pallas_call(
        _mm_body,
        grid=(NROW,),
        in_specs=[pl.BlockSpec((128, D), lambda i: (i, 0)),
                  pl.BlockSpec((D, D), lambda i: (0, 0))],
        out_specs=pl.BlockSpec((128, D), lambda i: (i, 0)),
        out_shape=jax.ShapeDtypeStruct((NPAD, D), jnp.float32),
    )(xpad, W)


def _layer_body(p_ref, xw_ref, d2_ref, b_ref, w_ref, o_ref, *, relu):
    h = (p_ref[0] + p_ref[1] + xw_ref[...] * d2_ref[0, 0][:, None]
         + b_ref[...])
    if relu:
        h = jnp.maximum(h, 0.0)
    o_ref[...] = jnp.dot(h, w_ref[...], preferred_element_type=jnp.float32)


def _layer(parts, xw, dinv2, b, W, relu):
    return pl.pallas_call(
        functools.partial(_layer_body, relu=relu),
        grid=(NROW,),
        in_specs=[pl.BlockSpec((NC, 128, D), lambda i: (0, i, 0)),
                  pl.BlockSpec((128, D), lambda i: (i, 0)),
                  pl.BlockSpec((1, 1, 128), lambda i: (i, 0, 0)),
                  pl.BlockSpec((1, D), lambda i: (0, 0)),
                  pl.BlockSpec((D, D), lambda i: (0, 0))],
        out_specs=pl.BlockSpec((128, D), lambda i: (i, 0)),
        out_shape=jax.ShapeDtypeStruct((NPAD, D), jnp.float32),
    )(parts, xw, dinv2, b.reshape(1, D), W)


def _final_body(p_ref, xw_ref, d2_ref, b_ref, wl_ref, bl_ref, ptr_ref, o_ref):
    i = pl.program_id(0)
    h = (p_ref[0] + p_ref[1] + xw_ref[...] * d2_ref[0, 0][:, None]
         + b_ref[...])
    t = jnp.dot(h, wl_ref[...], preferred_element_type=jnp.float32)
    onehot = (ptr_ref[0, 0][:, None]
              == lax.broadcasted_iota(jnp.int32, (1, NG), 1)
              ).astype(jnp.float32)
    contrib = jnp.dot(onehot.T, t, preferred_element_type=jnp.float32)

    @pl.when(i == 0)
    def _():
        o_ref[...] = jnp.broadcast_to(bl_ref[...], (NG, OUTD))

    o_ref[...] += contrib


def _final(parts, xw, dinv2, b, Wl, bl, ptr_pad):
    return pl.pallas_call(
        _final_body,
        grid=(NROW,),
        in_specs=[pl.BlockSpec((NC, 128, D), lambda i: (0, i, 0)),
                  pl.BlockSpec((128, D), lambda i: (i, 0)),
                  pl.BlockSpec((1, 1, 128), lambda i: (i, 0, 0)),
                  pl.BlockSpec((1, D), lambda i: (0, 0)),
                  pl.BlockSpec((D, OUTD), lambda i: (0, 0)),
                  pl.BlockSpec((1, OUTD), lambda i: (0, 0)),
                  pl.BlockSpec((1, 1, 128), lambda i: (i, 0, 0))],
        out_specs=pl.BlockSpec((NG, OUTD), lambda i: (0, 0)),
        out_shape=jax.ShapeDtypeStruct((NG, OUTD), jnp.float32),
    )(parts, xw, dinv2, b.reshape(1, D), Wl,
      bl.reshape(1, OUTD), ptr_pad.reshape(NROW, 1, 128))


def kernel(x, edge_index, edge_attr, ptr, W1, b1, W2, b2, W3, b3, Wl, bl):
    row = edge_index[0]
    col = edge_index[1]

    norm, dinv2 = _prep(row, col, edge_attr)
    dinv2 = dinv2.reshape(NROW, 1, 128)

    xpad = jnp.concatenate(
        [x, jnp.zeros((NPAD - N, D), jnp.float32)], axis=0)
    ptr_pad = jnp.concatenate(
        [ptr, jnp.full((NPAD - N,), NG, jnp.int32)]).reshape(NROW, 128)

    xw1 = _mm(xpad, W1)
    parts1 = _mp(xw1, row, col, norm)
    xw2 = _layer(parts1, xw1, dinv2, b1, W2, relu=False)
    parts2 = _mp(xw2, row, col, norm)
    xw3 = _layer(parts2, xw2, dinv2, b2, W3, relu=True)
    parts3 = _mp(xw3, row, col, norm)
    out = _final(parts3, xw3, dinv2, b3, Wl, bl, ptr_pad)
    return out


# speedup vs baseline: 12.1049x; 1.2390x over previous
import functools

import jax
import jax.numpy as jnp
from jax import lax
from jax.experimental import pallas as pl
from jax.experimental.pallas import tpu as pltpu
from jax.experimental.pallas import tpu_sc as plsc

N = 10000
E = 320000
D = 128
OUTD = 64
NG = 16

NC, NS, L = 2, 16, 16
NW = NC * NS
NPAD = 10240
NROW = 80
EPW = E // NW
EPT = E // NS
CH = 80
NCH = EPW // CH

_MESH = plsc.VectorSubcoreMesh(
    core_axis_name="c", subcore_axis_name="s", num_cores=NC, num_subcores=NS)


def _rsqrt16(d):
    i = plsc.bitcast(d, jnp.int32)
    y = plsc.bitcast(jnp.int32(0x5F3759DF) - (i >> 1), jnp.float32)
    for _ in range(3):
        y = y * (1.5 - 0.5 * d * y * y)
    return y


def _prep_body(row_hbm, col_hbm, ew_hbm,
               norm_hbm, dinv2_hbm,
               colb, ewb, rowb, normb, degacc, dsum, tmp, yloc, y2loc,
               sdeg_all, sdinv):
    c = lax.axis_index("c")
    s = lax.axis_index("s")
    wid = c * NS + s
    zeros16 = jnp.zeros((16,), jnp.float32)
    npt = NPAD // NS

    @pl.loop(0, NPAD // 16)
    def _(r):
        degacc[pl.ds(r * 16, 16)] = zeros16

    for half in range(2):
        off = s * EPT + half * EPW
        pltpu.sync_copy(col_hbm.at[pl.ds(off, EPW)], colb)
        pltpu.sync_copy(ew_hbm.at[pl.ds(off, EPW)], ewb)

        @pl.loop(0, EPW // 16)
        def _(i):
            c16 = colb[pl.ds(i * 16, 16)]
            w16 = ewb[pl.ds(i * 16, 16)]
            plsc.addupdate_scatter(degacc, [c16], w16)

    pltpu.sync_copy(degacc, sdeg_all.at[s])
    plsc.subcore_barrier()

    @pl.loop(0, npt // 16)
    def _(j):
        dsum[pl.ds(j * 16, 16)] = zeros16
    for p in range(NS):
        pltpu.sync_copy(sdeg_all.at[p, pl.ds(s * npt, npt)], tmp)

        @pl.loop(0, npt // 16)
        def _(j):
            sl = pl.ds(j * 16, 16)
            dsum[sl] = dsum[sl] + tmp[sl]

    @pl.loop(0, npt // 16)
    def _(j):
        sl = pl.ds(j * 16, 16)
        d = dsum[sl] + 1.0
        y = _rsqrt16(d)
        yloc[sl] = y
        y2loc[sl] = y * y

    pltpu.sync_copy(yloc, sdinv.at[pl.ds(s * npt, npt)])

    @pl.when(c == 0)
    def _():
        pltpu.sync_copy(y2loc, dinv2_hbm.at[pl.ds(s * npt, npt)])

    plsc.subcore_barrier()

    pltpu.sync_copy(sdinv, degacc)
    off = wid * EPW
    pltpu.sync_copy(row_hbm.at[pl.ds(off, EPW)], rowb)
    pltpu.sync_copy(col_hbm.at[pl.ds(off, EPW)], colb)
    pltpu.sync_copy(ew_hbm.at[pl.ds(off, EPW)], ewb)

    @pl.loop(0, EPW // 16)
    def _(i):
        sl = pl.ds(i * 16, 16)
        dr = plsc.load_gather(degacc, [rowb[sl]])
        dc = plsc.load_gather(degacc, [colb[sl]])
        normb[sl] = dr * dc * ewb[sl]

    pltpu.sync_copy(normb, norm_hbm.at[pl.ds(off, EPW)])


_prep = pl.kernel(
    _prep_body,
    out_type=(
        jax.ShapeDtypeStruct((E,), jnp.float32),
        jax.ShapeDtypeStruct((NPAD,), jnp.float32),
    ),
    mesh=_MESH,
    scratch_types=[
        pltpu.VMEM((EPW,), jnp.int32),
        pltpu.VMEM((EPW,), jnp.float32),
        pltpu.VMEM((EPW,), jnp.int32),
        pltpu.VMEM((EPW,), jnp.float32),
        pltpu.VMEM((NPAD,), jnp.float32),
        pltpu.VMEM((NPAD // NS,), jnp.float32),
        pltpu.VMEM((NPAD // NS,), jnp.float32),
        pltpu.VMEM((NPAD // NS,), jnp.float32),
        pltpu.VMEM((NPAD // NS,), jnp.float32),
        pltpu.VMEM_SHARED((NS, NPAD), jnp.float32),
        pltpu.VMEM_SHARED((NPAD,), jnp.float32),
    ],
    compiler_params=pltpu.CompilerParams(needs_layout_passes=False),
)


_QCH = (32, 32, 32, 29)
_QBASE = (0, 32, 64, 96)
_QMAX = 32


def _mp_body(xw_hbm, row_hbm, col4_hbm, norm_hbm,
             parts_hbm,
             rowb, normb, colb2, rows0, rows1, rows2,
             gs0, gs1, gs2, ss0, ss1, ss2,
             acc):
    c = lax.axis_index("c")
    s = lax.axis_index("s")
    wid = c * NS + s
    zeros16 = jnp.zeros((16,), jnp.float32)
    rowsb = (rows0, rows1, rows2)
    gsem = (gs0, gs1, gs2)
    ssem = (ss0, ss1, ss2)

    @pl.loop(0, CH)
    def _(r):
        for j in range(8):
            rows0[r, pl.ds(j * 16, 16)] = zeros16
    for k in range(8):
        pltpu.sync_copy(rows0, acc.at[pl.ds(s * 640 + k * 80, 80)])
    plsc.subcore_barrier()

    base = wid * EPW

    def gather_start(b, g):
        pltpu.async_copy(xw_hbm.at[rowb.at[pl.ds(g * CH, CH)]],
                         rowsb[b], gsem[b])

    def gather_wait(b, g):
        pltpu.make_async_copy(xw_hbm.at[rowb.at[pl.ds(g * CH, CH)]],
                              rowsb[b], gsem[b]).wait()

    def scat_start(b, g):
        pltpu.async_copy(rowsb[b], acc.at[colb2.at[g]], ssem[b], add=True)

    def scat_drain(b):
        pltpu.make_async_copy(rowsb[b], acc.at[colb2.at[0]], ssem[b]).wait()

    def scale(b, g):
        buf = rowsb[b]
        goff = g * CH
        zi = jnp.zeros((16,), jnp.int32)

        @pl.loop(0, 5)
        def _(grp):
            for e in range(16):
                ei = grp * 16 + e
                ne = plsc.load_gather(normb, [zi + (goff + ei)])
                for j in range(8):
                    sl = pl.ds(j * 16, 16)
                    buf[ei, sl] = buf[ei, sl] * ne

    for q in range(4):
        qn = _QCH[q]
        ne_q = qn * CH
        eoff = base + _QBASE[q] * CH
        pltpu.sync_copy(row_hbm.at[pl.ds(eoff, ne_q)],
                        rowb.at[pl.ds(0, ne_q)])
        pltpu.sync_copy(norm_hbm.at[pl.ds(eoff, ne_q)],
                        normb.at[pl.ds(0, ne_q)])
        pltpu.sync_copy(col4_hbm.at[wid, pl.ds(_QBASE[q], qn)],
                        colb2.at[pl.ds(0, qn)])

        gather_start(0, 0)
        mt = (qn - 2) // 3

        @pl.loop(0, mt)
        def _(gt):
            for k in range(3):
                g = gt * 3 + k
                gather_wait(k, g)
                scale(k, g)
                scat_start(k, g)

                @pl.when(g >= 2)
                def _():
                    scat_drain((k + 1) % 3)

                gather_start((k + 1) % 3, g + 1)

        for g in range(3 * mt, qn):
            b = g % 3
            gather_wait(b, g)
            scale(b, g)
            scat_start(b, g)
            if g >= 2:
                scat_drain((g - 2) % 3)
            if g + 1 < qn:
                gather_start((g + 1) % 3, g + 1)
        scat_drain((qn - 2) % 3)
        scat_drain((qn - 1) % 3)

    plsc.subcore_barrier()
    pltpu.sync_copy(acc.at[pl.ds(s * 640, 640)],
                    parts_hbm.at[c, pl.ds(s * 640, 640)])


_mp = pl.kernel(
    _mp_body,
    out_type=jax.ShapeDtypeStruct((NC, NPAD, 128), jnp.float32),
    mesh=_MESH,
    scratch_types=[
        pltpu.VMEM((_QMAX * CH,), jnp.int32),
        pltpu.VMEM((_QMAX * CH,), jnp.float32),
        pltpu.VMEM((_QMAX, CH), jnp.int32),
        pltpu.VMEM((CH, 128), jnp.float32),
        pltpu.VMEM((CH, 128), jnp.float32),
        pltpu.VMEM((CH, 128), jnp.float32),
        pltpu.SemaphoreType.DMA,
        pltpu.SemaphoreType.DMA,
        pltpu.SemaphoreType.DMA,
        pltpu.SemaphoreType.DMA,
        pltpu.SemaphoreType.DMA,
        pltpu.SemaphoreType.DMA,
        pltpu.VMEM_SHARED((NPAD, 128), jnp.float32),
    ],
    compiler_params=pltpu.CompilerParams(needs_layout_passes=False),
)


def _mm_body(x_ref, w_ref, o_ref):
    o_ref[...] = jnp.dot(x_ref[...], w_ref[...],
                         preferred_element_type=jnp.float32)


def _mm(xpad, W):
    return pl.pallas_call(
        _mm_body,
        grid=(NROW,),
        in_specs=[pl.BlockSpec((128, D), lambda i: (i, 0)),
                  pl.BlockSpec((D, D), lambda i: (0, 0))],
        out_specs=pl.BlockSpec((128, D), lambda i: (i, 0)),
        out_shape=jax.ShapeDtypeStruct((NPAD, D), jnp.float32),
    )(xpad, W)


def _layer_body(p_ref, xw_ref, d2_ref, b_ref, w_ref, o_ref, *, relu):
    h = (p_ref[0] + p_ref[1] + xw_ref[...] * d2_ref[0, 0][:, None]
         + b_ref[...])
    if relu:
        h = jnp.maximum(h, 0.0)
    o_ref[...] = jnp.dot(h, w_ref[...], preferred_element_type=jnp.float32)


def _layer(parts, xw, dinv2, b, W, relu):
    return pl.pallas_call(
        functools.partial(_layer_body, relu=relu),
        grid=(NROW,),
        in_specs=[pl.BlockSpec((NC, 128, D), lambda i: (0, i, 0)),
                  pl.BlockSpec((128, D), lambda i: (i, 0)),
                  pl.BlockSpec((1, 1, 128), lambda i: (i, 0, 0)),
                  pl.BlockSpec((1, D), lambda i: (0, 0)),
                  pl.BlockSpec((D, D), lambda i: (0, 0))],
        out_specs=pl.BlockSpec((128, D), lambda i: (i, 0)),
        out_shape=jax.ShapeDtypeStruct((NPAD, D), jnp.float32),
    )(parts, xw, dinv2, b.reshape(1, D), W)


def _final_body(p_ref, xw_ref, d2_ref, b_ref, wl_ref, bl_ref, ptr_ref, o_ref):
    i = pl.program_id(0)
    h = (p_ref[0] + p_ref[1] + xw_ref[...] * d2_ref[0, 0][:, None]
         + b_ref[...])
    t = jnp.dot(h, wl_ref[...], preferred_element_type=jnp.float32)
    onehot = (ptr_ref[0, 0][:, None]
              == lax.broadcasted_iota(jnp.int32, (1, NG), 1)
              ).astype(jnp.float32)
    contrib = jnp.dot(onehot.T, t, preferred_element_type=jnp.float32)

    @pl.when(i == 0)
    def _():
        o_ref[...] = jnp.broadcast_to(bl_ref[...], (NG, OUTD))

    o_ref[...] += contrib


def _final(parts, xw, dinv2, b, Wl, bl, ptr_pad):
    return pl.pallas_call(
        _final_body,
        grid=(NROW,),
        in_specs=[pl.BlockSpec((NC, 128, D), lambda i: (0, i, 0)),
                  pl.BlockSpec((128, D), lambda i: (i, 0)),
                  pl.BlockSpec((1, 1, 128), lambda i: (i, 0, 0)),
                  pl.BlockSpec((1, D), lambda i: (0, 0)),
                  pl.BlockSpec((D, OUTD), lambda i: (0, 0)),
                  pl.BlockSpec((1, OUTD), lambda i: (0, 0)),
                  pl.BlockSpec((1, 1, 128), lambda i: (i, 0, 0))],
        out_specs=pl.BlockSpec((NG, OUTD), lambda i: (0, 0)),
        out_shape=jax.ShapeDtypeStruct((NG, OUTD), jnp.float32),
    )(parts, xw, dinv2, b.reshape(1, D), Wl,
      bl.reshape(1, OUTD), ptr_pad.reshape(NROW, 1, 128))


def kernel(x, edge_index, edge_attr, ptr, W1, b1, W2, b2, W3, b3, Wl, bl):
    row = edge_index[0]
    col = edge_index[1]

    norm, dinv2 = _prep(row, col, edge_attr)
    dinv2 = dinv2.reshape(NROW, 1, 128)
    col4 = col.reshape(NW, NCH, CH)

    xpad = jnp.concatenate(
        [x, jnp.zeros((NPAD - N, D), jnp.float32)], axis=0)
    ptr_pad = jnp.concatenate(
        [ptr, jnp.full((NPAD - N,), NG, jnp.int32)]).reshape(NROW, 128)

    xw1 = _mm(xpad, W1)
    parts1 = _mp(xw1, row, col4, norm)
    xw2 = _layer(parts1, xw1, dinv2, b1, W2, relu=False)
    parts2 = _mp(xw2, row, col4, norm)
    xw3 = _layer(parts2, xw2, dinv2, b2, W3, relu=True)
    parts3 = _mp(xw3, row, col4, norm)
    out = _final(parts3, xw3, dinv2, b3, Wl, bl, ptr_pad)
    return out


# speedup vs baseline: 14.0272x; 1.1588x over previous
import functools

import jax
import jax.numpy as jnp
from jax import lax
from jax.experimental import pallas as pl
from jax.experimental.pallas import tpu as pltpu
from jax.experimental.pallas import tpu_sc as plsc

N = 10000
E = 320000
D = 128
OUTD = 64
NG = 16

NC, NS, L = 2, 16, 16
NW = NC * NS
NPAD = 10240
NROW = 80
EPW = E // NW
EPT = E // NS
CH = 80
NCH = EPW // CH

_MESH = plsc.VectorSubcoreMesh(
    core_axis_name="c", subcore_axis_name="s", num_cores=NC, num_subcores=NS)


def _rsqrt16(d):
    i = plsc.bitcast(d, jnp.int32)
    y = plsc.bitcast(jnp.int32(0x5F3759DF) - (i >> 1), jnp.float32)
    for _ in range(3):
        y = y * (1.5 - 0.5 * d * y * y)
    return y


def _prep_body(row_hbm, col_hbm, ew_hbm,
               norm_hbm, dinv2_hbm,
               colb, ewb, rowb, normb, degacc, dsum, tmp, yloc, y2loc,
               sdeg_all, sdinv):
    c = lax.axis_index("c")
    s = lax.axis_index("s")
    wid = c * NS + s
    zeros16 = jnp.zeros((16,), jnp.float32)
    npt = NPAD // NS

    @pl.loop(0, NPAD // 16)
    def _(r):
        degacc[pl.ds(r * 16, 16)] = zeros16

    for half in range(2):
        off = s * EPT + half * EPW
        pltpu.sync_copy(col_hbm.at[pl.ds(off, EPW)], colb)
        pltpu.sync_copy(ew_hbm.at[pl.ds(off, EPW)], ewb)

        @pl.loop(0, EPW // 16)
        def _(i):
            c16 = colb[pl.ds(i * 16, 16)]
            w16 = ewb[pl.ds(i * 16, 16)]
            plsc.addupdate_scatter(degacc, [c16], w16)

    pltpu.sync_copy(degacc, sdeg_all.at[s])
    plsc.subcore_barrier()

    @pl.loop(0, npt // 16)
    def _(j):
        dsum[pl.ds(j * 16, 16)] = zeros16
    for p in range(NS):
        pltpu.sync_copy(sdeg_all.at[p, pl.ds(s * npt, npt)], tmp)

        @pl.loop(0, npt // 16)
        def _(j):
            sl = pl.ds(j * 16, 16)
            dsum[sl] = dsum[sl] + tmp[sl]

    @pl.loop(0, npt // 16)
    def _(j):
        sl = pl.ds(j * 16, 16)
        d = dsum[sl] + 1.0
        y = _rsqrt16(d)
        yloc[sl] = y
        y2loc[sl] = y * y

    pltpu.sync_copy(yloc, sdinv.at[pl.ds(s * npt, npt)])

    @pl.when(c == 0)
    def _():
        pltpu.sync_copy(y2loc, dinv2_hbm.at[pl.ds(s * npt, npt)])

    plsc.subcore_barrier()

    pltpu.sync_copy(sdinv, degacc)
    off = wid * EPW
    pltpu.sync_copy(row_hbm.at[pl.ds(off, EPW)], rowb)
    pltpu.sync_copy(col_hbm.at[pl.ds(off, EPW)], colb)
    pltpu.sync_copy(ew_hbm.at[pl.ds(off, EPW)], ewb)

    @pl.loop(0, EPW // 16)
    def _(i):
        sl = pl.ds(i * 16, 16)
        dr = plsc.load_gather(degacc, [rowb[sl]])
        dc = plsc.load_gather(degacc, [colb[sl]])
        normb[sl] = dr * dc * ewb[sl]

    pltpu.sync_copy(normb, norm_hbm.at[pl.ds(off, EPW)])


_prep = pl.kernel(
    _prep_body,
    out_type=(
        jax.ShapeDtypeStruct((E,), jnp.float32),
        jax.ShapeDtypeStruct((NPAD,), jnp.float32),
    ),
    mesh=_MESH,
    scratch_types=[
        pltpu.VMEM((EPW,), jnp.int32),
        pltpu.VMEM((EPW,), jnp.float32),
        pltpu.VMEM((EPW,), jnp.int32),
        pltpu.VMEM((EPW,), jnp.float32),
        pltpu.VMEM((NPAD,), jnp.float32),
        pltpu.VMEM((NPAD // NS,), jnp.float32),
        pltpu.VMEM((NPAD // NS,), jnp.float32),
        pltpu.VMEM((NPAD // NS,), jnp.float32),
        pltpu.VMEM((NPAD // NS,), jnp.float32),
        pltpu.VMEM_SHARED((NS, NPAD), jnp.float32),
        pltpu.VMEM_SHARED((NPAD,), jnp.float32),
    ],
    compiler_params=pltpu.CompilerParams(needs_layout_passes=False),
)


_QCH = (32, 32, 32, 29)
_QBASE = (0, 32, 64, 96)
_QMAX = 32


def _mp_body(xw_hbm, row_hbm, col4_hbm, norm_hbm,
             parts_hbm,
             rowb, normb, colb2, rows0, rows1, rows2,
             gs0, gs1, gs2, ss0, ss1, ss2,
             acc):
    c = lax.axis_index("c")
    s = lax.axis_index("s")
    wid = c * NS + s
    zeros16 = jnp.zeros((16,), jnp.float32)
    rowsb = (rows0, rows1, rows2)
    gsem = (gs0, gs1, gs2)
    ssem = (ss0, ss1, ss2)

    @pl.loop(0, CH)
    def _(r):
        for j in range(8):
            rows0[r, pl.ds(j * 16, 16)] = zeros16
    for k in range(8):
        pltpu.sync_copy(rows0, acc.at[pl.ds(s * 640 + k * 80, 80)])
    plsc.subcore_barrier()

    base = wid * EPW

    def gather_start(b, g):
        pltpu.async_copy(xw_hbm.at[rowb.at[pl.ds(g * CH, CH)]],
                         rowsb[b], gsem[b])

    def gather_wait(b, g):
        pltpu.make_async_copy(xw_hbm.at[rowb.at[pl.ds(g * CH, CH)]],
                              rowsb[b], gsem[b]).wait()

    def scat_start(b, g):
        pltpu.async_copy(rowsb[b], acc.at[colb2.at[g]], ssem[b], add=True)

    def scat_drain(b):
        pltpu.make_async_copy(rowsb[b], acc.at[colb2.at[0]], ssem[b]).wait()

    def scale(b, g):
        buf = rowsb[b]
        goff = g * CH

        @pl.loop(0, 5)
        def _(grp):
            n16 = normb[pl.ds(goff + grp * 16, 16)]
            for e in range(16):
                ei = grp * 16 + e
                ne = jnp.take_along_axis(
                    n16, jnp.full((16,), e, jnp.int32), axis=0,
                    mode="promise_in_bounds")
                for j in range(8):
                    sl = pl.ds(j * 16, 16)
                    buf[ei, sl] = buf[ei, sl] * ne

    for q in range(4):
        qn = _QCH[q]
        ne_q = qn * CH
        eoff = base + _QBASE[q] * CH
        pltpu.sync_copy(row_hbm.at[pl.ds(eoff, ne_q)],
                        rowb.at[pl.ds(0, ne_q)])
        pltpu.sync_copy(norm_hbm.at[pl.ds(eoff, ne_q)],
                        normb.at[pl.ds(0, ne_q)])
        pltpu.sync_copy(col4_hbm.at[wid, pl.ds(_QBASE[q], qn)],
                        colb2.at[pl.ds(0, qn)])

        gather_start(0, 0)
        mt = (qn - 2) // 3

        @pl.loop(0, mt)
        def _(gt):
            for k in range(3):
                g = gt * 3 + k
                gather_wait(k, g)
                scale(k, g)
                scat_start(k, g)

                @pl.when(g >= 2)
                def _():
                    scat_drain((k + 1) % 3)

                gather_start((k + 1) % 3, g + 1)

        for g in range(3 * mt, qn):
            b = g % 3
            gather_wait(b, g)
            scale(b, g)
            scat_start(b, g)
            if g >= 2:
                scat_drain((g - 2) % 3)
            if g + 1 < qn:
                gather_start((g + 1) % 3, g + 1)
        scat_drain((qn - 2) % 3)
        scat_drain((qn - 1) % 3)

    plsc.subcore_barrier()
    pltpu.sync_copy(acc.at[pl.ds(s * 640, 640)],
                    parts_hbm.at[c, pl.ds(s * 640, 640)])


_mp = pl.kernel(
    _mp_body,
    out_type=jax.ShapeDtypeStruct((NC, NPAD, 128), jnp.float32),
    mesh=_MESH,
    scratch_types=[
        pltpu.VMEM((_QMAX * CH,), jnp.int32),
        pltpu.VMEM((_QMAX * CH,), jnp.float32),
        pltpu.VMEM((_QMAX, CH), jnp.int32),
        pltpu.VMEM((CH, 128), jnp.float32),
        pltpu.VMEM((CH, 128), jnp.float32),
        pltpu.VMEM((CH, 128), jnp.float32),
        pltpu.SemaphoreType.DMA,
        pltpu.SemaphoreType.DMA,
        pltpu.SemaphoreType.DMA,
        pltpu.SemaphoreType.DMA,
        pltpu.SemaphoreType.DMA,
        pltpu.SemaphoreType.DMA,
        pltpu.VMEM_SHARED((NPAD, 128), jnp.float32),
    ],
    compiler_params=pltpu.CompilerParams(needs_layout_passes=False),
)


def _mm_body(x_ref, w_ref, o_ref):
    o_ref[...] = jnp.dot(x_ref[...], w_ref[...],
                         preferred_element_type=jnp.float32)


def _mm(xpad, W):
    return pl.pallas_call(
        _mm_body,
        grid=(NROW,),
        in_specs=[pl.BlockSpec((128, D), lambda i: (i, 0)),
                  pl.BlockSpec((D, D), lambda i: (0, 0))],
        out_specs=pl.BlockSpec((128, D), lambda i: (i, 0)),
        out_shape=jax.ShapeDtypeStruct((NPAD, D), jnp.float32),
    )(xpad, W)


def _layer_body(p_ref, xw_ref, d2_ref, b_ref, w_ref, o_ref, *, relu):
    h = (p_ref[0] + p_ref[1] + xw_ref[...] * d2_ref[0, 0][:, None]
         + b_ref[...])
    if relu:
        h = jnp.maximum(h, 0.0)
    o_ref[...] = jnp.dot(h, w_ref[...], preferred_element_type=jnp.float32)


def _layer(parts, xw, dinv2, b, W, relu):
    return pl.pallas_call(
        functools.partial(_layer_body, relu=relu),
        grid=(NROW,),
        in_specs=[pl.BlockSpec((NC, 128, D), lambda i: (0, i, 0)),
                  pl.BlockSpec((128, D), lambda i: (i, 0)),
                  pl.BlockSpec((1, 1, 128), lambda i: (i, 0, 0)),
                  pl.BlockSpec((1, D), lambda i: (0, 0)),
                  pl.BlockSpec((D, D), lambda i: (0, 0))],
        out_specs=pl.BlockSpec((128, D), lambda i: (i, 0)),
        out_shape=jax.ShapeDtypeStruct((NPAD, D), jnp.float32),
    )(parts, xw, dinv2, b.reshape(1, D), W)


def _final_body(p_ref, xw_ref, d2_ref, b_ref, wl_ref, bl_ref, ptr_ref, o_ref):
    i = pl.program_id(0)
    h = (p_ref[0] + p_ref[1] + xw_ref[...] * d2_ref[0, 0][:, None]
         + b_ref[...])
    t = jnp.dot(h, wl_ref[...], preferred_element_type=jnp.float32)
    onehot = (ptr_ref[0, 0][:, None]
              == lax.broadcasted_iota(jnp.int32, (1, NG), 1)
              ).astype(jnp.float32)
    contrib = jnp.dot(onehot.T, t, preferred_element_type=jnp.float32)

    @pl.when(i == 0)
    def _():
        o_ref[...] = jnp.broadcast_to(bl_ref[...], (NG, OUTD))

    o_ref[...] += contrib


def _final(parts, xw, dinv2, b, Wl, bl, ptr_pad):
    return pl.pallas_call(
        _final_body,
        grid=(NROW,),
        in_specs=[pl.BlockSpec((NC, 128, D), lambda i: (0, i, 0)),
                  pl.BlockSpec((128, D), lambda i: (i, 0)),
                  pl.BlockSpec((1, 1, 128), lambda i: (i, 0, 0)),
                  pl.BlockSpec((1, D), lambda i: (0, 0)),
                  pl.BlockSpec((D, OUTD), lambda i: (0, 0)),
                  pl.BlockSpec((1, OUTD), lambda i: (0, 0)),
                  pl.BlockSpec((1, 1, 128), lambda i: (i, 0, 0))],
        out_specs=pl.BlockSpec((NG, OUTD), lambda i: (0, 0)),
        out_shape=jax.ShapeDtypeStruct((NG, OUTD), jnp.float32),
    )(parts, xw, dinv2, b.reshape(1, D), Wl,
      bl.reshape(1, OUTD), ptr_pad.reshape(NROW, 1, 128))


def kernel(x, edge_index, edge_attr, ptr, W1, b1, W2, b2, W3, b3, Wl, bl):
    row = edge_index[0]
    col = edge_index[1]

    norm, dinv2 = _prep(row, col, edge_attr)
    dinv2 = dinv2.reshape(NROW, 1, 128)
    col4 = col.reshape(NW, NCH, CH)

    xpad = jnp.concatenate(
        [x, jnp.zeros((NPAD - N, D), jnp.float32)], axis=0)
    ptr_pad = jnp.concatenate(
        [ptr, jnp.full((NPAD - N,), NG, jnp.int32)]).reshape(NROW, 128)

    xw1 = _mm(xpad, W1)
    parts1 = _mp(xw1, row, col4, norm)
    xw2 = _layer(parts1, xw1, dinv2, b1, W2, relu=False)
    parts2 = _mp(xw2, row, col4, norm)
    xw3 = _layer(parts2, xw2, dinv2, b2, W3, relu=True)
    parts3 = _mp(xw3, row, col4, norm)
    out = _final(parts3, xw3, dinv2, b3, Wl, bl, ptr_pad)
    return out


# speedup vs baseline: 16.6812x; 1.1892x over previous
import functools

import jax
import jax.numpy as jnp
from jax import lax
from jax.experimental import pallas as pl
from jax.experimental.pallas import tpu as pltpu
from jax.experimental.pallas import tpu_sc as plsc

N = 10000
E = 320000
D = 128
OUTD = 64
NG = 16

NC, NS, L = 2, 16, 16
NW = NC * NS
NPAD = 10240
NROW = 80
EPW = E // NW
EPT = E // NS
CH = 80
NCH = EPW // CH

_MESH = plsc.VectorSubcoreMesh(
    core_axis_name="c", subcore_axis_name="s", num_cores=NC, num_subcores=NS)


def _rsqrt16(d):
    i = plsc.bitcast(d, jnp.int32)
    y = plsc.bitcast(jnp.int32(0x5F3759DF) - (i >> 1), jnp.float32)
    for _ in range(3):
        y = y * (1.5 - 0.5 * d * y * y)
    return y


def _prep_body(row_hbm, col_hbm, ew_hbm,
               norm_hbm, dinv2_hbm,
               colb, ewb, rowb, normb, degacc, dsum, tmp, yloc, y2loc,
               sdeg_all, sdinv):
    c = lax.axis_index("c")
    s = lax.axis_index("s")
    wid = c * NS + s
    zeros16 = jnp.zeros((16,), jnp.float32)
    npt = NPAD // NS

    @pl.loop(0, NPAD // 16)
    def _(r):
        degacc[pl.ds(r * 16, 16)] = zeros16

    for half in range(2):
        off = s * EPT + half * EPW
        pltpu.sync_copy(col_hbm.at[pl.ds(off, EPW)], colb)
        pltpu.sync_copy(ew_hbm.at[pl.ds(off, EPW)], ewb)

        @pl.loop(0, EPW // 16)
        def _(i):
            c16 = colb[pl.ds(i * 16, 16)]
            w16 = ewb[pl.ds(i * 16, 16)]
            plsc.addupdate_scatter(degacc, [c16], w16)

    pltpu.sync_copy(degacc, sdeg_all.at[s])
    plsc.subcore_barrier()

    @pl.loop(0, npt // 16)
    def _(j):
        dsum[pl.ds(j * 16, 16)] = zeros16
    for p in range(NS):
        pltpu.sync_copy(sdeg_all.at[p, pl.ds(s * npt, npt)], tmp)

        @pl.loop(0, npt // 16)
        def _(j):
            sl = pl.ds(j * 16, 16)
            dsum[sl] = dsum[sl] + tmp[sl]

    @pl.loop(0, npt // 16)
    def _(j):
        sl = pl.ds(j * 16, 16)
        d = dsum[sl] + 1.0
        y = _rsqrt16(d)
        yloc[sl] = y
        y2loc[sl] = y * y

    pltpu.sync_copy(yloc, sdinv.at[pl.ds(s * npt, npt)])

    @pl.when(c == 0)
    def _():
        pltpu.sync_copy(y2loc, dinv2_hbm.at[pl.ds(s * npt, npt)])

    plsc.subcore_barrier()

    pltpu.sync_copy(sdinv, degacc)
    off = wid * EPW
    pltpu.sync_copy(row_hbm.at[pl.ds(off, EPW)], rowb)
    pltpu.sync_copy(col_hbm.at[pl.ds(off, EPW)], colb)
    pltpu.sync_copy(ew_hbm.at[pl.ds(off, EPW)], ewb)

    @pl.loop(0, EPW // 16)
    def _(i):
        sl = pl.ds(i * 16, 16)
        dr = plsc.load_gather(degacc, [rowb[sl]])
        dc = plsc.load_gather(degacc, [colb[sl]])
        normb[sl] = dr * dc * ewb[sl]

    pltpu.sync_copy(normb, norm_hbm.at[pl.ds(off, EPW)])


_prep = pl.kernel(
    _prep_body,
    out_type=(
        jax.ShapeDtypeStruct((E,), jnp.float32),
        jax.ShapeDtypeStruct((NPAD,), jnp.float32),
    ),
    mesh=_MESH,
    scratch_types=[
        pltpu.VMEM((EPW,), jnp.int32),
        pltpu.VMEM((EPW,), jnp.float32),
        pltpu.VMEM((EPW,), jnp.int32),
        pltpu.VMEM((EPW,), jnp.float32),
        pltpu.VMEM((NPAD,), jnp.float32),
        pltpu.VMEM((NPAD // NS,), jnp.float32),
        pltpu.VMEM((NPAD // NS,), jnp.float32),
        pltpu.VMEM((NPAD // NS,), jnp.float32),
        pltpu.VMEM((NPAD // NS,), jnp.float32),
        pltpu.VMEM_SHARED((NS, NPAD), jnp.float32),
        pltpu.VMEM_SHARED((NPAD,), jnp.float32),
    ],
    compiler_params=pltpu.CompilerParams(needs_layout_passes=False),
)


_QCH = (32, 32, 32, 29)
_QBASE = (0, 32, 64, 96)
_QMAX = 32


def _mp_body(xw_hbm, row_hbm, col4_hbm, norm_hbm,
             parts_hbm,
             rowb, normb, colb2, rows0, rows1, rows2,
             gs0, gs1, gs2, ss0, ss1, ss2,
             acc):
    c = lax.axis_index("c")
    s = lax.axis_index("s")
    wid = c * NS + s
    zeros16 = jnp.zeros((16,), jnp.float32)
    rowsb = (rows0, rows1, rows2)
    gsem = (gs0, gs1, gs2)
    ssem = (ss0, ss1, ss2)

    @pl.loop(0, CH)
    def _(r):
        for j in range(8):
            rows0[r, pl.ds(j * 16, 16)] = zeros16
    for k in range(8):
        pltpu.sync_copy(rows0, acc.at[pl.ds(s * 640 + k * 80, 80)])
    plsc.subcore_barrier()

    base = wid * EPW

    def gather_start(b, g):
        pltpu.async_copy(xw_hbm.at[rowb.at[pl.ds(g * CH, CH)]],
                         rowsb[b], gsem[b])

    def gather_wait(b, g):
        pltpu.make_async_copy(xw_hbm.at[rowb.at[pl.ds(g * CH, CH)]],
                              rowsb[b], gsem[b]).wait()

    def scat_start(b, g):
        pltpu.async_copy(rowsb[b], acc.at[colb2.at[g]], ssem[b], add=True)

    def scat_drain(b):
        pltpu.make_async_copy(rowsb[b], acc.at[colb2.at[0]], ssem[b]).wait()

    def scale(b, g):
        buf = rowsb[b]
        goff = g * CH

        @pl.loop(0, 5)
        def _(grp):
            n16 = normb[pl.ds(goff + grp * 16, 16)]
            for e in range(16):
                ei = grp * 16 + e
                ne = jnp.take_along_axis(
                    n16, jnp.full((16,), e, jnp.int32), axis=0,
                    mode="promise_in_bounds")
                for j in range(8):
                    sl = pl.ds(j * 16, 16)
                    buf[ei, sl] = buf[ei, sl] * ne

    for q in range(4):
        qn = _QCH[q]
        ne_q = qn * CH
        eoff = base + _QBASE[q] * CH
        pltpu.sync_copy(row_hbm.at[pl.ds(eoff, ne_q)],
                        rowb.at[pl.ds(0, ne_q)])
        pltpu.sync_copy(norm_hbm.at[pl.ds(eoff, ne_q)],
                        normb.at[pl.ds(0, ne_q)])
        pltpu.sync_copy(col4_hbm.at[wid, pl.ds(_QBASE[q], qn)],
                        colb2.at[pl.ds(0, qn)])

        gather_start(0, 0)
        mt = (qn - 2) // 3

        @pl.loop(0, mt)
        def _(gt):
            for k in range(3):
                g = gt * 3 + k
                gather_wait(k, g)

                @pl.when(g >= 2)
                def _():
                    scat_drain((k + 1) % 3)

                gather_start((k + 1) % 3, g + 1)
                scale(k, g)
                scat_start(k, g)

        for g in range(3 * mt, qn):
            b = g % 3
            gather_wait(b, g)
            if g >= 2:
                scat_drain((g - 2) % 3)
            if g + 1 < qn:
                gather_start((g + 1) % 3, g + 1)
            scale(b, g)
            scat_start(b, g)
        scat_drain((qn - 2) % 3)
        scat_drain((qn - 1) % 3)

    plsc.subcore_barrier()
    pltpu.sync_copy(acc.at[pl.ds(s * 640, 640)],
                    parts_hbm.at[c, pl.ds(s * 640, 640)])


_mp = pl.kernel(
    _mp_body,
    out_type=jax.ShapeDtypeStruct((NC, NPAD, 128), jnp.float32),
    mesh=_MESH,
    scratch_types=[
        pltpu.VMEM((_QMAX * CH,), jnp.int32),
        pltpu.VMEM((_QMAX * CH,), jnp.float32),
        pltpu.VMEM((_QMAX, CH), jnp.int32),
        pltpu.VMEM((CH, 128), jnp.float32),
        pltpu.VMEM((CH, 128), jnp.float32),
        pltpu.VMEM((CH, 128), jnp.float32),
        pltpu.SemaphoreType.DMA,
        pltpu.SemaphoreType.DMA,
        pltpu.SemaphoreType.DMA,
        pltpu.SemaphoreType.DMA,
        pltpu.SemaphoreType.DMA,
        pltpu.SemaphoreType.DMA,
        pltpu.VMEM_SHARED((NPAD, 128), jnp.float32),
    ],
    compiler_params=pltpu.CompilerParams(needs_layout_passes=False),
)


def _mm_body(x_ref, w_ref, o_ref):
    o_ref[...] = jnp.dot(x_ref[...], w_ref[...],
                         preferred_element_type=jnp.float32)


def _mm(xpad, W):
    return pl.pallas_call(
        _mm_body,
        grid=(NROW,),
        in_specs=[pl.BlockSpec((128, D), lambda i: (i, 0)),
                  pl.BlockSpec((D, D), lambda i: (0, 0))],
        out_specs=pl.BlockSpec((128, D), lambda i: (i, 0)),
        out_shape=jax.ShapeDtypeStruct((NPAD, D), jnp.float32),
    )(xpad, W)


def _layer_body(p_ref, xw_ref, d2_ref, b_ref, w_ref, o_ref, *, relu):
    h = (p_ref[0] + p_ref[1] + xw_ref[...] * d2_ref[0, 0][:, None]
         + b_ref[...])
    if relu:
        h = jnp.maximum(h, 0.0)
    o_ref[...] = jnp.dot(h, w_ref[...], preferred_element_type=jnp.float32)


def _layer(parts, xw, dinv2, b, W, relu):
    return pl.pallas_call(
        functools.partial(_layer_body, relu=relu),
        grid=(NROW,),
        in_specs=[pl.BlockSpec((NC, 128, D), lambda i: (0, i, 0)),
                  pl.BlockSpec((128, D), lambda i: (i, 0)),
                  pl.BlockSpec((1, 1, 128), lambda i: (i, 0, 0)),
                  pl.BlockSpec((1, D), lambda i: (0, 0)),
                  pl.BlockSpec((D, D), lambda i: (0, 0))],
        out_specs=pl.BlockSpec((128, D), lambda i: (i, 0)),
        out_shape=jax.ShapeDtypeStruct((NPAD, D), jnp.float32),
    )(parts, xw, dinv2, b.reshape(1, D), W)


def _final_body(p_ref, xw_ref, d2_ref, b_ref, wl_ref, bl_ref, ptr_ref, o_ref):
    i = pl.program_id(0)
    h = (p_ref[0] + p_ref[1] + xw_ref[...] * d2_ref[0, 0][:, None]
         + b_ref[...])
    t = jnp.dot(h, wl_ref[...], preferred_element_type=jnp.float32)
    onehot = (ptr_ref[0, 0][:, None]
              == lax.broadcasted_iota(jnp.int32, (1, NG), 1)
              ).astype(jnp.float32)
    contrib = jnp.dot(onehot.T, t, preferred_element_type=jnp.float32)

    @pl.when(i == 0)
    def _():
        o_ref[...] = jnp.broadcast_to(bl_ref[...], (NG, OUTD))

    o_ref[...] += contrib


def _final(parts, xw, dinv2, b, Wl, bl, ptr_pad):
    return pl.pallas_call(
        _final_body,
        grid=(NROW,),
        in_specs=[pl.BlockSpec((NC, 128, D), lambda i: (0, i, 0)),
                  pl.BlockSpec((128, D), lambda i: (i, 0)),
                  pl.BlockSpec((1, 1, 128), lambda i: (i, 0, 0)),
                  pl.BlockSpec((1, D), lambda i: (0, 0)),
                  pl.BlockSpec((D, OUTD), lambda i: (0, 0)),
                  pl.BlockSpec((1, OUTD), lambda i: (0, 0)),
                  pl.BlockSpec((1, 1, 128), lambda i: (i, 0, 0))],
        out_specs=pl.BlockSpec((NG, OUTD), lambda i: (0, 0)),
        out_shape=jax.ShapeDtypeStruct((NG, OUTD), jnp.float32),
    )(parts, xw, dinv2, b.reshape(1, D), Wl,
      bl.reshape(1, OUTD), ptr_pad.reshape(NROW, 1, 128))


def kernel(x, edge_index, edge_attr, ptr, W1, b1, W2, b2, W3, b3, Wl, bl):
    row = edge_index[0]
    col = edge_index[1]

    norm, dinv2 = _prep(row, col, edge_attr)
    dinv2 = dinv2.reshape(NROW, 1, 128)
    col4 = col.reshape(NW, NCH, CH)

    xpad = jnp.concatenate(
        [x, jnp.zeros((NPAD - N, D), jnp.float32)], axis=0)
    ptr_pad = jnp.concatenate(
        [ptr, jnp.full((NPAD - N,), NG, jnp.int32)]).reshape(NROW, 128)

    xw1 = _mm(xpad, W1)
    parts1 = _mp(xw1, row, col4, norm)
    xw2 = _layer(parts1, xw1, dinv2, b1, W2, relu=False)
    parts2 = _mp(xw2, row, col4, norm)
    xw3 = _layer(parts2, xw2, dinv2, b2, W3, relu=True)
    parts3 = _mp(xw3, row, col4, norm)
    out = _final(parts3, xw3, dinv2, b3, Wl, bl, ptr_pad)
    return out


# speedup vs baseline: 17.2240x; 1.0325x over previous
import functools

import jax
import jax.numpy as jnp
from jax import lax
from jax.experimental import pallas as pl
from jax.experimental.pallas import tpu as pltpu
from jax.experimental.pallas import tpu_sc as plsc

N = 10000
E = 320000
D = 128
OUTD = 64
NG = 16

NC, NS, L = 2, 16, 16
NW = NC * NS
NPAD = 10240
NROW = 80
EPW = E // NW
EPT = E // NS
CH = 80
NCH = EPW // CH

_MESH = plsc.VectorSubcoreMesh(
    core_axis_name="c", subcore_axis_name="s", num_cores=NC, num_subcores=NS)


def _rsqrt16(d):
    i = plsc.bitcast(d, jnp.int32)
    y = plsc.bitcast(jnp.int32(0x5F3759DF) - (i >> 1), jnp.float32)
    for _ in range(3):
        y = y * (1.5 - 0.5 * d * y * y)
    return y


def _prep_body(row_hbm, col_hbm, ew_hbm,
               norm_hbm, dinv2_hbm,
               colb, ewb, rowb, normb, degacc, dsum, tmp, yloc, y2loc,
               sdeg_all, sdinv):
    c = lax.axis_index("c")
    s = lax.axis_index("s")
    wid = c * NS + s
    zeros16 = jnp.zeros((16,), jnp.float32)
    npt = NPAD // NS

    @pl.loop(0, NPAD // 16)
    def _(r):
        degacc[pl.ds(r * 16, 16)] = zeros16

    for half in range(2):
        off = s * EPT + half * EPW
        pltpu.sync_copy(col_hbm.at[pl.ds(off, EPW)], colb)
        pltpu.sync_copy(ew_hbm.at[pl.ds(off, EPW)], ewb)

        @pl.loop(0, EPW // 16)
        def _(i):
            c16 = colb[pl.ds(i * 16, 16)]
            w16 = ewb[pl.ds(i * 16, 16)]
            plsc.addupdate_scatter(degacc, [c16], w16)

    pltpu.sync_copy(degacc, sdeg_all.at[s])
    plsc.subcore_barrier()

    @pl.loop(0, npt // 16)
    def _(j):
        dsum[pl.ds(j * 16, 16)] = zeros16
    for p in range(NS):
        pltpu.sync_copy(sdeg_all.at[p, pl.ds(s * npt, npt)], tmp)

        @pl.loop(0, npt // 16)
        def _(j):
            sl = pl.ds(j * 16, 16)
            dsum[sl] = dsum[sl] + tmp[sl]

    @pl.loop(0, npt // 16)
    def _(j):
        sl = pl.ds(j * 16, 16)
        d = dsum[sl] + 1.0
        y = _rsqrt16(d)
        yloc[sl] = y
        y2loc[sl] = y * y

    pltpu.sync_copy(yloc, sdinv.at[pl.ds(s * npt, npt)])

    @pl.when(c == 0)
    def _():
        pltpu.sync_copy(y2loc, dinv2_hbm.at[pl.ds(s * npt, npt)])

    plsc.subcore_barrier()

    pltpu.sync_copy(sdinv, degacc)
    off = wid * EPW
    pltpu.sync_copy(row_hbm.at[pl.ds(off, EPW)], rowb)
    pltpu.sync_copy(col_hbm.at[pl.ds(off, EPW)], colb)
    pltpu.sync_copy(ew_hbm.at[pl.ds(off, EPW)], ewb)

    @pl.loop(0, EPW // 16)
    def _(i):
        sl = pl.ds(i * 16, 16)
        dr = plsc.load_gather(degacc, [rowb[sl]])
        dc = plsc.load_gather(degacc, [colb[sl]])
        normb[sl] = dr * dc * ewb[sl]

    pltpu.sync_copy(normb, norm_hbm.at[pl.ds(off, EPW)])


_prep = pl.kernel(
    _prep_body,
    out_type=(
        jax.ShapeDtypeStruct((E,), jnp.float32),
        jax.ShapeDtypeStruct((NPAD,), jnp.float32),
    ),
    mesh=_MESH,
    scratch_types=[
        pltpu.VMEM((EPW,), jnp.int32),
        pltpu.VMEM((EPW,), jnp.float32),
        pltpu.VMEM((EPW,), jnp.int32),
        pltpu.VMEM((EPW,), jnp.float32),
        pltpu.VMEM((NPAD,), jnp.float32),
        pltpu.VMEM((NPAD // NS,), jnp.float32),
        pltpu.VMEM((NPAD // NS,), jnp.float32),
        pltpu.VMEM((NPAD // NS,), jnp.float32),
        pltpu.VMEM((NPAD // NS,), jnp.float32),
        pltpu.VMEM_SHARED((NS, NPAD), jnp.float32),
        pltpu.VMEM_SHARED((NPAD,), jnp.float32),
    ],
    compiler_params=pltpu.CompilerParams(needs_layout_passes=False),
)


_QCH = (32, 32, 32, 29)
_QBASE = (0, 32, 64, 96)
_QMAX = 32


def _mp_body(xw_hbm, row_hbm, col4_hbm, norm_hbm,
             parts_hbm,
             rowb, normb, colb2, rows0, rows1, rows2,
             gs0, gs1, gs2, hs0, hs1, hs2, ss0, ss1, ss2,
             acc):
    c = lax.axis_index("c")
    s = lax.axis_index("s")
    wid = c * NS + s
    zeros16 = jnp.zeros((16,), jnp.float32)
    rowsb = (rows0, rows1, rows2)
    gsem = (gs0, gs1, gs2)
    hsem = (hs0, hs1, hs2)
    ssem = (ss0, ss1, ss2)

    @pl.loop(0, CH)
    def _(r):
        for j in range(8):
            rows0[r, pl.ds(j * 16, 16)] = zeros16
    for k in range(8):
        pltpu.sync_copy(rows0, acc.at[pl.ds(s * 640 + k * 80, 80)])
    plsc.subcore_barrier()

    base = wid * EPW

    def gather_start(b, g):
        pltpu.async_copy(xw_hbm.at[rowb.at[pl.ds(g * CH, 48)]],
                         rowsb[b].at[pl.ds(0, 48)], gsem[b])
        pltpu.async_copy(xw_hbm.at[rowb.at[pl.ds(g * CH + 48, 32)]],
                         rowsb[b].at[pl.ds(48, 32)], hsem[b])

    def gather_wait_a(b, g):
        pltpu.make_async_copy(xw_hbm.at[rowb.at[pl.ds(g * CH, 48)]],
                              rowsb[b].at[pl.ds(0, 48)], gsem[b]).wait()

    def gather_wait_b(b, g):
        pltpu.make_async_copy(xw_hbm.at[rowb.at[pl.ds(g * CH + 48, 32)]],
                              rowsb[b].at[pl.ds(48, 32)], hsem[b]).wait()

    def scat_start(b, g):
        pltpu.async_copy(rowsb[b], acc.at[colb2.at[g]], ssem[b], add=True)

    def scat_drain(b):
        pltpu.make_async_copy(rowsb[b], acc.at[colb2.at[0]], ssem[b]).wait()

    def scale_part(b, g, lo, hi):
        buf = rowsb[b]
        goff = g * CH

        @pl.loop(lo, hi)
        def _(grp):
            n16 = normb[pl.ds(goff + grp * 16, 16)]
            for e in range(16):
                ei = grp * 16 + e
                ne = jnp.take_along_axis(
                    n16, jnp.full((16,), e, jnp.int32), axis=0,
                    mode="promise_in_bounds")
                for j in range(8):
                    sl = pl.ds(j * 16, 16)
                    buf[ei, sl] = buf[ei, sl] * ne

    for q in range(4):
        qn = _QCH[q]
        ne_q = qn * CH
        eoff = base + _QBASE[q] * CH
        pltpu.sync_copy(row_hbm.at[pl.ds(eoff, ne_q)],
                        rowb.at[pl.ds(0, ne_q)])
        pltpu.sync_copy(norm_hbm.at[pl.ds(eoff, ne_q)],
                        normb.at[pl.ds(0, ne_q)])
        pltpu.sync_copy(col4_hbm.at[wid, pl.ds(_QBASE[q], qn)],
                        colb2.at[pl.ds(0, qn)])

        gather_start(0, 0)
        mt = (qn - 2) // 3

        @pl.loop(0, mt)
        def _(gt):
            for k in range(3):
                g = gt * 3 + k
                gather_wait_a(k, g)

                @pl.when(g >= 2)
                def _():
                    scat_drain((k + 1) % 3)

                gather_start((k + 1) % 3, g + 1)
                scale_part(k, g, 0, 3)
                gather_wait_b(k, g)
                scale_part(k, g, 3, 5)
                scat_start(k, g)

        for g in range(3 * mt, qn):
            b = g % 3
            gather_wait_a(b, g)
            if g >= 2:
                scat_drain((g - 2) % 3)
            if g + 1 < qn:
                gather_start((g + 1) % 3, g + 1)
            scale_part(b, g, 0, 3)
            gather_wait_b(b, g)
            scale_part(b, g, 3, 5)
            scat_start(b, g)
        scat_drain((qn - 2) % 3)
        scat_drain((qn - 1) % 3)

    plsc.subcore_barrier()
    pltpu.sync_copy(acc.at[pl.ds(s * 640, 640)],
                    parts_hbm.at[c, pl.ds(s * 640, 640)])


_mp = pl.kernel(
    _mp_body,
    out_type=jax.ShapeDtypeStruct((NC, NPAD, 128), jnp.float32),
    mesh=_MESH,
    scratch_types=[
        pltpu.VMEM((_QMAX * CH,), jnp.int32),
        pltpu.VMEM((_QMAX * CH,), jnp.float32),
        pltpu.VMEM((_QMAX, CH), jnp.int32),
        pltpu.VMEM((CH, 128), jnp.float32),
        pltpu.VMEM((CH, 128), jnp.float32),
        pltpu.VMEM((CH, 128), jnp.float32),
        pltpu.SemaphoreType.DMA,
        pltpu.SemaphoreType.DMA,
        pltpu.SemaphoreType.DMA,
        pltpu.SemaphoreType.DMA,
        pltpu.SemaphoreType.DMA,
        pltpu.SemaphoreType.DMA,
        pltpu.SemaphoreType.DMA,
        pltpu.SemaphoreType.DMA,
        pltpu.SemaphoreType.DMA,
        pltpu.VMEM_SHARED((NPAD, 128), jnp.float32),
    ],
    compiler_params=pltpu.CompilerParams(needs_layout_passes=False),
)


def _mm_body(x_ref, w_ref, o_ref):
    o_ref[...] = jnp.dot(x_ref[...], w_ref[...],
                         preferred_element_type=jnp.float32)


def _mm(xpad, W):
    return pl.pallas_call(
        _mm_body,
        grid=(NROW,),
        in_specs=[pl.BlockSpec((128, D), lambda i: (i, 0)),
                  pl.BlockSpec((D, D), lambda i: (0, 0))],
        out_specs=pl.BlockSpec((128, D), lambda i: (i, 0)),
        out_shape=jax.ShapeDtypeStruct((NPAD, D), jnp.float32),
    )(xpad, W)


def _layer_body(p_ref, xw_ref, d2_ref, b_ref, w_ref, o_ref, *, relu):
    h = (p_ref[0] + p_ref[1] + xw_ref[...] * d2_ref[0, 0][:, None]
         + b_ref[...])
    if relu:
        h = jnp.maximum(h, 0.0)
    o_ref[...] = jnp.dot(h, w_ref[...], preferred_element_type=jnp.float32)


def _layer(parts, xw, dinv2, b, W, relu):
    return pl.pallas_call(
        functools.partial(_layer_body, relu=relu),
        grid=(NROW,),
        in_specs=[pl.BlockSpec((NC, 128, D), lambda i: (0, i, 0)),
                  pl.BlockSpec((128, D), lambda i: (i, 0)),
                  pl.BlockSpec((1, 1, 128), lambda i: (i, 0, 0)),
                  pl.BlockSpec((1, D), lambda i: (0, 0)),
                  pl.BlockSpec((D, D), lambda i: (0, 0))],
        out_specs=pl.BlockSpec((128, D), lambda i: (i, 0)),
        out_shape=jax.ShapeDtypeStruct((NPAD, D), jnp.float32),
    )(parts, xw, dinv2, b.reshape(1, D), W)


def _final_body(p_ref, xw_ref, d2_ref, b_ref, wl_ref, bl_ref, ptr_ref, o_ref):
    i = pl.program_id(0)
    h = (p_ref[0] + p_ref[1] + xw_ref[...] * d2_ref[0, 0][:, None]
         + b_ref[...])
    t = jnp.dot(h, wl_ref[...], preferred_element_type=jnp.float32)
    onehot = (ptr_ref[0, 0][:, None]
              == lax.broadcasted_iota(jnp.int32, (1, NG), 1)
              ).astype(jnp.float32)
    contrib = jnp.dot(onehot.T, t, preferred_element_type=jnp.float32)

    @pl.when(i == 0)
    def _():
        o_ref[...] = jnp.broadcast_to(bl_ref[...], (NG, OUTD))

    o_ref[...] += contrib


def _final(parts, xw, dinv2, b, Wl, bl, ptr_pad):
    return pl.pallas_call(
        _final_body,
        grid=(NROW,),
        in_specs=[pl.BlockSpec((NC, 128, D), lambda i: (0, i, 0)),
                  pl.BlockSpec((128, D), lambda i: (i, 0)),
                  pl.BlockSpec((1, 1, 128), lambda i: (i, 0, 0)),
                  pl.BlockSpec((1, D), lambda i: (0, 0)),
                  pl.BlockSpec((D, OUTD), lambda i: (0, 0)),
                  pl.BlockSpec((1, OUTD), lambda i: (0, 0)),
                  pl.BlockSpec((1, 1, 128), lambda i: (i, 0, 0))],
        out_specs=pl.BlockSpec((NG, OUTD), lambda i: (0, 0)),
        out_shape=jax.ShapeDtypeStruct((NG, OUTD), jnp.float32),
    )(parts, xw, dinv2, b.reshape(1, D), Wl,
      bl.reshape(1, OUTD), ptr_pad.reshape(NROW, 1, 128))


def kernel(x, edge_index, edge_attr, ptr, W1, b1, W2, b2, W3, b3, Wl, bl):
    row = edge_index[0]
    col = edge_index[1]

    norm, dinv2 = _prep(row, col, edge_attr)
    dinv2 = dinv2.reshape(NROW, 1, 128)
    col4 = col.reshape(NW, NCH, CH)

    xpad = jnp.concatenate(
        [x, jnp.zeros((NPAD - N, D), jnp.float32)], axis=0)
    ptr_pad = jnp.concatenate(
        [ptr, jnp.full((NPAD - N,), NG, jnp.int32)]).reshape(NROW, 128)

    xw1 = _mm(xpad, W1)
    parts1 = _mp(xw1, row, col4, norm)
    xw2 = _layer(parts1, xw1, dinv2, b1, W2, relu=False)
    parts2 = _mp(xw2, row, col4, norm)
    xw3 = _layer(parts2, xw2, dinv2, b2, W3, relu=True)
    parts3 = _mp(xw3, row, col4, norm)
    out = _final(parts3, xw3, dinv2, b3, Wl, bl, ptr_pad)
    return out


# speedup vs baseline: 17.6898x; 1.0270x over previous
import functools

import jax
import jax.numpy as jnp
from jax import lax
from jax.experimental import pallas as pl
from jax.experimental.pallas import tpu as pltpu
from jax.experimental.pallas import tpu_sc as plsc

N = 10000
E = 320000
D = 128
OUTD = 64
NG = 16

NC, NS, L = 2, 16, 16
NW = NC * NS
NPAD = 10240
NROW = 80
EPW = E // NW
EPT = E // NS
CH = 80
NCH = EPW // CH

_MESH = plsc.VectorSubcoreMesh(
    core_axis_name="c", subcore_axis_name="s", num_cores=NC, num_subcores=NS)


def _rsqrt16(d):
    i = plsc.bitcast(d, jnp.int32)
    y = plsc.bitcast(jnp.int32(0x5F3759DF) - (i >> 1), jnp.float32)
    for _ in range(3):
        y = y * (1.5 - 0.5 * d * y * y)
    return y


def _prep_body(row_hbm, col_hbm, ew_hbm,
               norm_hbm, dinv2_hbm,
               colb, ewb, rowb, normb, degacc, dsum, tmp, yloc, y2loc,
               sdeg_all, sdinv):
    c = lax.axis_index("c")
    s = lax.axis_index("s")
    wid = c * NS + s
    zeros16 = jnp.zeros((16,), jnp.float32)
    npt = NPAD // NS

    @pl.loop(0, NPAD // 16)
    def _(r):
        degacc[pl.ds(r * 16, 16)] = zeros16

    for half in range(2):
        off = s * EPT + half * EPW
        pltpu.sync_copy(col_hbm.at[pl.ds(off, EPW)], colb)
        pltpu.sync_copy(ew_hbm.at[pl.ds(off, EPW)], ewb)

        @pl.loop(0, EPW // 16)
        def _(i):
            c16 = colb[pl.ds(i * 16, 16)]
            w16 = ewb[pl.ds(i * 16, 16)]
            plsc.addupdate_scatter(degacc, [c16], w16)

    pltpu.sync_copy(degacc, sdeg_all.at[s])
    plsc.subcore_barrier()

    @pl.loop(0, npt // 16)
    def _(j):
        dsum[pl.ds(j * 16, 16)] = zeros16
    for p in range(NS):
        pltpu.sync_copy(sdeg_all.at[p, pl.ds(s * npt, npt)], tmp)

        @pl.loop(0, npt // 16)
        def _(j):
            sl = pl.ds(j * 16, 16)
            dsum[sl] = dsum[sl] + tmp[sl]

    @pl.loop(0, npt // 16)
    def _(j):
        sl = pl.ds(j * 16, 16)
        d = dsum[sl] + 1.0
        y = _rsqrt16(d)
        yloc[sl] = y
        y2loc[sl] = y * y

    pltpu.sync_copy(yloc, sdinv.at[pl.ds(s * npt, npt)])

    @pl.when(c == 0)
    def _():
        pltpu.sync_copy(y2loc, dinv2_hbm.at[pl.ds(s * npt, npt)])

    plsc.subcore_barrier()

    pltpu.sync_copy(sdinv, degacc)
    off = wid * EPW
    pltpu.sync_copy(row_hbm.at[pl.ds(off, EPW)], rowb)
    pltpu.sync_copy(col_hbm.at[pl.ds(off, EPW)], colb)
    pltpu.sync_copy(ew_hbm.at[pl.ds(off, EPW)], ewb)

    @pl.loop(0, EPW // 16)
    def _(i):
        sl = pl.ds(i * 16, 16)
        dr = plsc.load_gather(degacc, [rowb[sl]])
        dc = plsc.load_gather(degacc, [colb[sl]])
        normb[sl] = dr * dc * ewb[sl]

    pltpu.sync_copy(normb, norm_hbm.at[pl.ds(off, EPW)])


_prep = pl.kernel(
    _prep_body,
    out_type=(
        jax.ShapeDtypeStruct((E,), jnp.float32),
        jax.ShapeDtypeStruct((NPAD,), jnp.float32),
    ),
    mesh=_MESH,
    scratch_types=[
        pltpu.VMEM((EPW,), jnp.int32),
        pltpu.VMEM((EPW,), jnp.float32),
        pltpu.VMEM((EPW,), jnp.int32),
        pltpu.VMEM((EPW,), jnp.float32),
        pltpu.VMEM((NPAD,), jnp.float32),
        pltpu.VMEM((NPAD // NS,), jnp.float32),
        pltpu.VMEM((NPAD // NS,), jnp.float32),
        pltpu.VMEM((NPAD // NS,), jnp.float32),
        pltpu.VMEM((NPAD // NS,), jnp.float32),
        pltpu.VMEM_SHARED((NS, NPAD), jnp.float32),
        pltpu.VMEM_SHARED((NPAD,), jnp.float32),
    ],
    compiler_params=pltpu.CompilerParams(needs_layout_passes=False),
)


_QCH = (32, 32, 32, 29)
_QBASE = (0, 32, 64, 96)
_QMAX = 32


def _mp_body(xw_hbm, row_hbm, col4_hbm, norm_hbm,
             parts_hbm,
             rowb, normb, colb2, rows0, rows1, rows2,
             gs0, gs1, gs2, hs0, hs1, hs2, ss0, ss1, ss2,
             acc):
    c = lax.axis_index("c")
    s = lax.axis_index("s")
    wid = c * NS + s
    zeros16 = jnp.zeros((16,), jnp.float32)
    rowsb = (rows0, rows1, rows2)
    gsem = (gs0, gs1, gs2)
    hsem = (hs0, hs1, hs2)
    ssem = (ss0, ss1, ss2)

    base = wid * EPW

    def preload_start(q, sem):
        qn = _QCH[q]
        ne_q = qn * CH
        eoff = base + _QBASE[q] * CH
        pltpu.async_copy(row_hbm.at[pl.ds(eoff, ne_q)],
                         rowb.at[pl.ds(0, ne_q)], sem)
        pltpu.async_copy(norm_hbm.at[pl.ds(eoff, ne_q)],
                         normb.at[pl.ds(0, ne_q)], sem)
        pltpu.async_copy(col4_hbm.at[wid, pl.ds(_QBASE[q], qn)],
                         colb2.at[pl.ds(0, qn)], sem)

    def preload_drain(q, sem):
        qn = _QCH[q]
        ne_q = qn * CH
        eoff = base + _QBASE[q] * CH
        pltpu.make_async_copy(row_hbm.at[pl.ds(eoff, ne_q)],
                              rowb.at[pl.ds(0, ne_q)], sem).wait()
        pltpu.make_async_copy(norm_hbm.at[pl.ds(eoff, ne_q)],
                              normb.at[pl.ds(0, ne_q)], sem).wait()
        pltpu.make_async_copy(col4_hbm.at[wid, pl.ds(_QBASE[q], qn)],
                              colb2.at[pl.ds(0, qn)], sem).wait()

    preload_start(0, ss0)

    @pl.loop(0, CH)
    def _(r):
        for j in range(8):
            rows0[r, pl.ds(j * 16, 16)] = zeros16
    for k in range(8):
        pltpu.async_copy(rows0, acc.at[pl.ds(s * 640 + k * 80, 80)], ss1)
    for k in range(8):
        pltpu.make_async_copy(rows0, acc.at[pl.ds(s * 640 + k * 80, 80)],
                              ss1).wait()
    plsc.subcore_barrier()

    def gather_start(b, g):
        pltpu.async_copy(xw_hbm.at[rowb.at[pl.ds(g * CH, 48)]],
                         rowsb[b].at[pl.ds(0, 48)], gsem[b])
        pltpu.async_copy(xw_hbm.at[rowb.at[pl.ds(g * CH + 48, 32)]],
                         rowsb[b].at[pl.ds(48, 32)], hsem[b])

    def gather_wait_a(b, g):
        pltpu.make_async_copy(xw_hbm.at[rowb.at[pl.ds(g * CH, 48)]],
                              rowsb[b].at[pl.ds(0, 48)], gsem[b]).wait()

    def gather_wait_b(b, g):
        pltpu.make_async_copy(xw_hbm.at[rowb.at[pl.ds(g * CH + 48, 32)]],
                              rowsb[b].at[pl.ds(48, 32)], hsem[b]).wait()

    def scat_start(b, g):
        pltpu.async_copy(rowsb[b], acc.at[colb2.at[g]], ssem[b], add=True)

    def scat_drain(b):
        pltpu.make_async_copy(rowsb[b], acc.at[colb2.at[0]], ssem[b]).wait()

    def scale_part(b, g, lo, hi):
        buf = rowsb[b]
        goff = g * CH

        @pl.loop(lo, hi)
        def _(grp):
            n16 = normb[pl.ds(goff + grp * 16, 16)]
            for e in range(16):
                ei = grp * 16 + e
                ne = jnp.take_along_axis(
                    n16, jnp.full((16,), e, jnp.int32), axis=0,
                    mode="promise_in_bounds")
                for j in range(8):
                    sl = pl.ds(j * 16, 16)
                    buf[ei, sl] = buf[ei, sl] * ne

    for q in range(4):
        qn = _QCH[q]
        if q == 0:
            preload_drain(0, ss0)
        else:
            preload_start(q, ss0)
            preload_drain(q, ss0)

        gather_start(0, 0)
        mt = (qn - 2) // 3

        @pl.loop(0, mt)
        def _(gt):
            for k in range(3):
                g = gt * 3 + k
                gather_wait_a(k, g)

                @pl.when(g >= 2)
                def _():
                    scat_drain((k + 1) % 3)

                gather_start((k + 1) % 3, g + 1)
                scale_part(k, g, 0, 3)
                gather_wait_b(k, g)
                scale_part(k, g, 3, 5)
                scat_start(k, g)

        for g in range(3 * mt, qn):
            b = g % 3
            gather_wait_a(b, g)
            if g >= 2:
                scat_drain((g - 2) % 3)
            if g + 1 < qn:
                gather_start((g + 1) % 3, g + 1)
            scale_part(b, g, 0, 3)
            gather_wait_b(b, g)
            scale_part(b, g, 3, 5)
            scat_start(b, g)
        scat_drain((qn - 2) % 3)
        scat_drain((qn - 1) % 3)

    plsc.subcore_barrier()
    pltpu.sync_copy(acc.at[pl.ds(s * 640, 640)],
                    parts_hbm.at[c, pl.ds(s * 640, 640)])


_mp = pl.kernel(
    _mp_body,
    out_type=jax.ShapeDtypeStruct((NC, NPAD, 128), jnp.float32),
    mesh=_MESH,
    scratch_types=[
        pltpu.VMEM((_QMAX * CH,), jnp.int32),
        pltpu.VMEM((_QMAX * CH,), jnp.float32),
        pltpu.VMEM((_QMAX, CH), jnp.int32),
        pltpu.VMEM((CH, 128), jnp.float32),
        pltpu.VMEM((CH, 128), jnp.float32),
        pltpu.VMEM((CH, 128), jnp.float32),
        pltpu.SemaphoreType.DMA,
        pltpu.SemaphoreType.DMA,
        pltpu.SemaphoreType.DMA,
        pltpu.SemaphoreType.DMA,
        pltpu.SemaphoreType.DMA,
        pltpu.SemaphoreType.DMA,
        pltpu.SemaphoreType.DMA,
        pltpu.SemaphoreType.DMA,
        pltpu.SemaphoreType.DMA,
        pltpu.VMEM_SHARED((NPAD, 128), jnp.float32),
    ],
    compiler_params=pltpu.CompilerParams(needs_layout_passes=False),
)


def _mm_body(x_ref, w_ref, o_ref):
    o_ref[...] = jnp.dot(x_ref[...], w_ref[...],
                         preferred_element_type=jnp.float32)


def _mm(xpad, W):
    return pl.pallas_call(
        _mm_body,
        grid=(NROW,),
        in_specs=[pl.BlockSpec((128, D), lambda i: (i, 0)),
                  pl.BlockSpec((D, D), lambda i: (0, 0))],
        out_specs=pl.BlockSpec((128, D), lambda i: (i, 0)),
        out_shape=jax.ShapeDtypeStruct((NPAD, D), jnp.float32),
    )(xpad, W)


def _layer_body(p_ref, xw_ref, d2_ref, b_ref, w_ref, o_ref, *, relu):
    h = (p_ref[0] + p_ref[1] + xw_ref[...] * d2_ref[0, 0][:, None]
         + b_ref[...])
    if relu:
        h = jnp.maximum(h, 0.0)
    o_ref[...] = jnp.dot(h, w_ref[...], preferred_element_type=jnp.float32)


def _layer(parts, xw, dinv2, b, W, relu):
    return pl.pallas_call(
        functools.partial(_layer_body, relu=relu),
        grid=(NROW,),
        in_specs=[pl.BlockSpec((NC, 128, D), lambda i: (0, i, 0)),
                  pl.BlockSpec((128, D), lambda i: (i, 0)),
                  pl.BlockSpec((1, 1, 128), lambda i: (i, 0, 0)),
                  pl.BlockSpec((1, D), lambda i: (0, 0)),
                  pl.BlockSpec((D, D), lambda i: (0, 0))],
        out_specs=pl.BlockSpec((128, D), lambda i: (i, 0)),
        out_shape=jax.ShapeDtypeStruct((NPAD, D), jnp.float32),
    )(parts, xw, dinv2, b.reshape(1, D), W)


def _final_body(p_ref, xw_ref, d2_ref, b_ref, wl_ref, bl_ref, ptr_ref, o_ref):
    i = pl.program_id(0)
    h = (p_ref[0] + p_ref[1] + xw_ref[...] * d2_ref[0, 0][:, None]
         + b_ref[...])
    t = jnp.dot(h, wl_ref[...], preferred_element_type=jnp.float32)
    onehot = (ptr_ref[0, 0][:, None]
              == lax.broadcasted_iota(jnp.int32, (1, NG), 1)
              ).astype(jnp.float32)
    contrib = jnp.dot(onehot.T, t, preferred_element_type=jnp.float32)

    @pl.when(i == 0)
    def _():
        o_ref[...] = jnp.broadcast_to(bl_ref[...], (NG, OUTD))

    o_ref[...] += contrib


def _final(parts, xw, dinv2, b, Wl, bl, ptr_pad):
    return pl.pallas_call(
        _final_body,
        grid=(NROW,),
        in_specs=[pl.BlockSpec((NC, 128, D), lambda i: (0, i, 0)),
                  pl.BlockSpec((128, D), lambda i: (i, 0)),
                  pl.BlockSpec((1, 1, 128), lambda i: (i, 0, 0)),
                  pl.BlockSpec((1, D), lambda i: (0, 0)),
                  pl.BlockSpec((D, OUTD), lambda i: (0, 0)),
                  pl.BlockSpec((1, OUTD), lambda i: (0, 0)),
                  pl.BlockSpec((1, 1, 128), lambda i: (i, 0, 0))],
        out_specs=pl.BlockSpec((NG, OUTD), lambda i: (0, 0)),
        out_shape=jax.ShapeDtypeStruct((NG, OUTD), jnp.float32),
    )(parts, xw, dinv2, b.reshape(1, D), Wl,
      bl.reshape(1, OUTD), ptr_pad.reshape(NROW, 1, 128))


def kernel(x, edge_index, edge_attr, ptr, W1, b1, W2, b2, W3, b3, Wl, bl):
    row = edge_index[0]
    col = edge_index[1]

    norm, dinv2 = _prep(row, col, edge_attr)
    dinv2 = dinv2.reshape(NROW, 1, 128)
    col4 = col.reshape(NW, NCH, CH)

    xpad = jnp.concatenate(
        [x, jnp.zeros((NPAD - N, D), jnp.float32)], axis=0)
    ptr_pad = jnp.concatenate(
        [ptr, jnp.full((NPAD - N,), NG, jnp.int32)]).reshape(NROW, 128)

    xw1 = _mm(xpad, W1)
    parts1 = _mp(xw1, row, col4, norm)
    xw2 = _layer(parts1, xw1, dinv2, b1, W2, relu=False)
    parts2 = _mp(xw2, row, col4, norm)
    xw3 = _layer(parts2, xw2, dinv2, b2, W3, relu=True)
    parts3 = _mp(xw3, row, col4, norm)
    out = _final(parts3, xw3, dinv2, b3, Wl, bl, ptr_pad)
    return out
